# baseline (speedup 1.0000x reference)
import functools
import math

import jax
import jax.numpy as jnp
from jax import lax
from jax.experimental import pallas as pl
from jax.experimental.pallas import tpu as pltpu
from jax.experimental.pallas import tpu_sc as plsc

D_MODEL = 64
SCALE = math.sqrt(D_MODEL)
NC = 2
NS = 16
NW = NC * NS
L = 16
G = 128
NBUF = 4
SEQ = 200
BATCH = 4096
TB = BATCH // G


def _emb_body(idx_hbm, table_hbm, out_hbm, idx_v, gbufs, obufs, gsems, osems):
    wid = lax.axis_index("s") * NC + lax.axis_index("c")
    b_per_w = idx_hbm.shape[0] // NW
    ngroups = b_per_w // G
    base = wid * b_per_w

    pltpu.sync_copy(idx_hbm.at[pl.ds(base, b_per_w)], idx_v)

    iota = lax.iota(jnp.int32, L)
    td_base = lax.shift_right_logical(iota, 3)
    inner_base = (iota & 7) << 7

    g0 = wid * ngroups

    def gstart(g, b):
        pltpu.make_async_copy(
            table_hbm.at[idx_v.at[pl.ds(g * G, G)]], gbufs[b], gsems[b]
        ).start()

    def gwait(b):
        pltpu.make_async_copy(
            table_hbm.at[idx_v.at[pl.ds(0, G)]], gbufs[b], gsems[b]
        ).wait()

    def ostart(g, b):
        gg = g0 + g
        s = gg // TB
        tb = gg % TB
        pltpu.make_async_copy(
            obufs[b], out_hbm.at[s, pl.ds(0, 8), tb], osems[b]
        ).start()

    def owait(b):
        pltpu.make_async_copy(
            obufs[b], out_hbm.at[0, pl.ds(0, 8), 0], osems[b]
        ).wait()

    def scale_transpose(b):
        def rows(i, c):
            inner = inner_base + i
            for j in range(D_MODEL // L):
                v = gbufs[b][i, pl.ds(j * L, L)] * SCALE
                plsc.store_scatter(obufs[b], [2 * j + td_base, inner], v)
            return c
        lax.fori_loop(0, G, rows, 0)

    for b in range(NBUF):
        gstart(b, b)

    for b in range(NBUF):
        gwait(b)
        scale_transpose(b)
        ostart(b, b)
        gstart(b + NBUF, b)

    def outer(o, c):
        for b in range(NBUF):
            g = o * NBUF + b
            gwait(b)
            owait(b)
            scale_transpose(b)
            ostart(g, b)
            gstart(g + NBUF, b)
        return c

    lax.fori_loop(1, ngroups // NBUF - 1, outer, 0)

    for b in range(NBUF):
        g = ngroups - NBUF + b
        gwait(b)
        owait(b)
        scale_transpose(b)
        ostart(g, b)

    for b in range(NBUF):
        owait(b)


def kernel(x, table):
    B = x.shape[0] * x.shape[1]
    assert x.shape == (BATCH, SEQ) and table.shape[1] == D_MODEL
    xf = x.T.reshape(B).astype(jnp.int32)
    mesh = plsc.VectorSubcoreMesh(core_axis_name="c", subcore_axis_name="s")
    b_per_w = B // NW
    assert B % NW == 0 and b_per_w % (G * NBUF) == 0
    run = functools.partial(
        pl.kernel,
        mesh=mesh,
        out_type=jax.ShapeDtypeStruct((SEQ, D_MODEL // 8, TB, 8 * G), jnp.float32),
        scratch_types=[
            pltpu.VMEM((b_per_w,), jnp.int32),
            [pltpu.VMEM((G, D_MODEL), jnp.float32) for _ in range(NBUF)],
            [pltpu.VMEM((8, 8 * G), jnp.float32) for _ in range(NBUF)],
            [pltpu.SemaphoreType.DMA for _ in range(NBUF)],
            [pltpu.SemaphoreType.DMA for _ in range(NBUF)],
        ],
        compiler_params=pltpu.CompilerParams(use_tc_tiling_on_sc=False, needs_layout_passes=False),
    )(_emb_body)
    out4 = run(xf, table)
    out = (
        out4.reshape(SEQ, D_MODEL // 8, TB, 8, G)
        .transpose(2, 4, 0, 1, 3)
        .reshape(BATCH, SEQ, D_MODEL)
    )
    return out


# speedup vs baseline: 1.5703x; 1.5703x over previous
import functools
import math

import jax
import jax.numpy as jnp
from jax import lax
from jax.experimental import pallas as pl
from jax.experimental.pallas import tpu as pltpu
from jax.experimental.pallas import tpu_sc as plsc

D_MODEL = 64
SCALE = math.sqrt(D_MODEL)
NC = 2
NS = 16
NW = NC * NS
L = 16
G = 128
NBUF = 4
ROW_UNROLL = 4
PADB = 133
SEQ = 200
BATCH = 4096
TB = BATCH // G


def _emb_body(idx_hbm, table_hbm, out_hbm, idx_v, gbufs, obufs, gsems, osems):
    wid = lax.axis_index("s") * NC + lax.axis_index("c")
    b_per_w = idx_hbm.shape[0] // NW
    ngroups = b_per_w // G
    base = wid * b_per_w

    pltpu.sync_copy(idx_hbm.at[pl.ds(base, b_per_w)], idx_v)

    iota = lax.iota(jnp.int32, L)
    td_base = lax.shift_right_logical(iota, 3)
    ds_base = iota & 7

    g0 = wid * ngroups

    def gstart(g, b):
        pltpu.make_async_copy(
            table_hbm.at[idx_v.at[pl.ds(g * G, G)]], gbufs[b], gsems[b]
        ).start()

    def gwait(b):
        pltpu.make_async_copy(
            table_hbm.at[idx_v.at[pl.ds(0, G)]], gbufs[b], gsems[b]
        ).wait()

    def ostart(g, b):
        gg = g0 + g
        s = gg // TB
        tb = gg % TB
        pltpu.make_async_copy(
            obufs[b].at[pl.ds(0, 8), pl.ds(0, 8), pl.ds(0, G)],
            out_hbm.at[s, pl.ds(0, 8), tb, pl.ds(0, 8), pl.ds(0, G)],
            osems[b],
        ).start()

    def owait(b):
        pltpu.make_async_copy(
            obufs[b].at[pl.ds(0, 8), pl.ds(0, 8), pl.ds(0, G)],
            out_hbm.at[0, pl.ds(0, 8), 0, pl.ds(0, 8), pl.ds(0, G)],
            osems[b],
        ).wait()

    def scale_transpose(b):
        def rows(i, c):
            for r in range(ROW_UNROLL):
                row = i * ROW_UNROLL + r
                bsplat = jnp.full((L,), row, dtype=jnp.int32)
                for j in range(D_MODEL // L):
                    v = gbufs[b][row, pl.ds(j * L, L)] * SCALE
                    plsc.store_scatter(
                        obufs[b], [2 * j + td_base, ds_base, bsplat], v
                    )
            return c
        lax.fori_loop(0, G // ROW_UNROLL, rows, 0)

    for b in range(NBUF):
        gstart(b, b)

    for b in range(NBUF):
        gwait(b)
        scale_transpose(b)
        ostart(b, b)
        gstart(b + NBUF, b)

    def outer(o, c):
        for b in range(NBUF):
            g = o * NBUF + b
            gwait(b)
            owait(b)
            scale_transpose(b)
            ostart(g, b)
            gstart(g + NBUF, b)
        return c

    lax.fori_loop(1, ngroups // NBUF - 1, outer, 0)

    for b in range(NBUF):
        g = ngroups - NBUF + b
        gwait(b)
        owait(b)
        scale_transpose(b)
        ostart(g, b)

    for b in range(NBUF):
        owait(b)


def kernel(x, table):
    B = x.shape[0] * x.shape[1]
    assert x.shape == (BATCH, SEQ) and table.shape[1] == D_MODEL
    xf = x.T.reshape(B).astype(jnp.int32)
    mesh = plsc.VectorSubcoreMesh(core_axis_name="c", subcore_axis_name="s")
    b_per_w = B // NW
    assert B % NW == 0 and b_per_w % (G * NBUF) == 0
    run = functools.partial(
        pl.kernel,
        mesh=mesh,
        out_type=jax.ShapeDtypeStruct((SEQ, D_MODEL // 8, TB, 8, G), jnp.float32),
        scratch_types=[
            pltpu.VMEM((b_per_w,), jnp.int32),
            [pltpu.VMEM((G, D_MODEL), jnp.float32) for _ in range(NBUF)],
            [pltpu.VMEM((8, 8, PADB), jnp.float32) for _ in range(NBUF)],
            [pltpu.SemaphoreType.DMA for _ in range(NBUF)],
            [pltpu.SemaphoreType.DMA for _ in range(NBUF)],
        ],
        compiler_params=pltpu.CompilerParams(use_tc_tiling_on_sc=False, needs_layout_passes=False),
    )(_emb_body)
    out5 = run(xf, table)
    return out5.transpose(2, 4, 0, 1, 3).reshape(BATCH, SEQ, D_MODEL)
